# baseline (speedup 1.0000x reference)
import functools

import jax
import jax.numpy as jnp
from jax import lax
from jax.experimental import pallas as pl
from jax.experimental.pallas import tpu as pltpu
from jax.experimental.pallas import tpu_sc as plsc


def _ln(x, g, b):
    m = jnp.mean(x, axis=-1, keepdims=True)
    v = jnp.mean((x - m) * (x - m), axis=-1, keepdims=True)
    return (x - m) / jnp.sqrt(v + 1e-5) * g + b


def _row(v):
    return v.reshape(1, -1)


_CH = 80
_KBUF = 5


@functools.lru_cache(maxsize=None)
def _make_spmm(n, e, d, de=0):
    pre = de > 0
    dh = d // 2
    epw = e // 16
    ch = _CH
    nchunk = epw // ch
    k = 5
    ngroup = nchunk // k
    rpt = n // 16
    mesh = plsc.VectorSubcoreMesh(core_axis_name="c", subcore_axis_name="s")

    def body(xlin_hbm, src_hbm, dst_hbm, zero_hbm, *rest):
        if pre:
            (ea_hbm, ones_hbm, zero16_hbm, zero8_hbm,
             out_hbm, ea_out, dg_out,
             idx_s, idx_d, rows, ebuf, ones, acc_sh, acc_ea, acc_dg,
             gsem, ssem, psem, qsem) = rest
        else:
            out_hbm, idx_s, idx_d, rows, acc_sh, gsem, ssem = rest
        cid = lax.axis_index("c")
        sid = lax.axis_index("s")
        c1 = pltpu.async_copy(src_hbm.at[pl.ds(sid * nchunk, nchunk), :],
                              idx_s, gsem.at[0])
        c2 = pltpu.async_copy(dst_hbm.at[pl.ds(sid * nchunk, nchunk), :],
                              idx_d, gsem.at[1])
        c3 = pltpu.async_copy(zero_hbm, acc_sh.at[pl.ds(sid * rpt, rpt)],
                              gsem.at[2])
        if pre:
            @pl.when(cid == 0)
            def _():
                pltpu.sync_copy(zero16_hbm, acc_ea.at[pl.ds(sid * rpt, rpt)])

            @pl.when(cid == 1)
            def _():
                pltpu.sync_copy(zero8_hbm, acc_dg.at[pl.ds(sid * rpt, rpt)])
                pltpu.sync_copy(ones_hbm, ones)
        c1.wait()
        c2.wait()
        c3.wait()
        plsc.subcore_barrier()
        half = xlin_hbm.at[cid]

        def group(g, carry):
            base = g * k
            for j in range(k):
                @pl.when(g > 0)
                def _():
                    pltpu.make_async_copy(
                        rows.at[j], acc_sh.at[idx_d.at[base + j]],
                        ssem.at[j]).wait()
                if pre:
                    @pl.when(jnp.logical_and(g > 0, cid == 0))
                    def _():
                        pltpu.make_async_copy(
                            ebuf.at[j], acc_ea.at[idx_d.at[base + j]],
                            qsem.at[j]).wait()

                    @pl.when(jnp.logical_and(g > 0, cid == 1))
                    def _():
                        pltpu.make_async_copy(
                            ones, acc_dg.at[idx_d.at[base + j]],
                            qsem.at[j]).wait()
                pltpu.async_copy(half.at[idx_s.at[base + j]],
                                 rows.at[j], gsem.at[j])
                if pre:
                    @pl.when(cid == 0)
                    def _():
                        pltpu.async_copy(
                            ea_hbm.at[pl.ds((sid * nchunk + base + j) * ch,
                                            ch), :],
                            ebuf.at[j], psem.at[j])
            for j in range(k):
                pltpu.make_async_copy(half.at[idx_s.at[base + j]],
                                      rows.at[j], gsem.at[j]).wait()
                pltpu.async_copy(rows.at[j], acc_sh.at[idx_d.at[base + j]],
                                 ssem.at[j], add=True)
                if pre:
                    @pl.when(cid == 0)
                    def _():
                        pltpu.make_async_copy(
                            ea_hbm.at[pl.ds((sid * nchunk + base + j) * ch,
                                            ch), :],
                            ebuf.at[j], psem.at[j]).wait()
                        pltpu.async_copy(ebuf.at[j],
                                         acc_ea.at[idx_d.at[base + j]],
                                         qsem.at[j], add=True)

                    @pl.when(cid == 1)
                    def _():
                        pltpu.async_copy(ones,
                                         acc_dg.at[idx_d.at[base + j]],
                                         qsem.at[j], add=True)
            return carry

        lax.fori_loop(0, ngroup, group, 0)
        for j in range(k):
            pltpu.make_async_copy(rows.at[j], acc_sh.at[idx_d.at[j]],
                                  ssem.at[j]).wait()
            if pre:
                @pl.when(cid == 0)
                def _():
                    pltpu.make_async_copy(ebuf.at[j], acc_ea.at[idx_d.at[j]],
                                          qsem.at[j]).wait()

                @pl.when(cid == 1)
                def _():
                    pltpu.make_async_copy(ones, acc_dg.at[idx_d.at[j]],
                                          qsem.at[j]).wait()
        plsc.subcore_barrier()
        pltpu.sync_copy(acc_sh.at[pl.ds(sid * rpt, rpt)],
                        out_hbm.at[cid, pl.ds(sid * rpt, rpt)])
        if pre:
            @pl.when(cid == 0)
            def _():
                pltpu.sync_copy(acc_ea.at[pl.ds(sid * rpt, rpt)],
                                ea_out.at[pl.ds(sid * rpt, rpt)])

            @pl.when(cid == 1)
            def _():
                pltpu.sync_copy(acc_dg.at[pl.ds(sid * rpt, rpt)],
                                dg_out.at[pl.ds(sid * rpt, rpt)])

    out_type = jax.ShapeDtypeStruct((2, n, dh), jnp.float32)
    scratch = [
        pltpu.VMEM((nchunk, ch), jnp.int32),
        pltpu.VMEM((nchunk, ch), jnp.int32),
        pltpu.VMEM((k, ch, dh), jnp.float32),
    ]
    if pre:
        out_type = (out_type,
                    jax.ShapeDtypeStruct((n, de), jnp.float32),
                    jax.ShapeDtypeStruct((n, 8), jnp.float32))
        scratch += [pltpu.VMEM((k, ch, de), jnp.float32),
                    pltpu.VMEM((ch, 8), jnp.float32)]
    scratch += [pltpu.VMEM_SHARED((n, dh), jnp.float32)]
    if pre:
        scratch += [pltpu.VMEM_SHARED((n, de), jnp.float32),
                    pltpu.VMEM_SHARED((n, 8), jnp.float32)]
    scratch += [pltpu.SemaphoreType.DMA((k,)),
                pltpu.SemaphoreType.DMA((k,))]
    if pre:
        scratch += [pltpu.SemaphoreType.DMA((k,)),
                    pltpu.SemaphoreType.DMA((k,))]

    return pl.kernel(
        body,
        out_type=out_type,
        mesh=mesh,
        compiler_params=pltpu.CompilerParams(use_tc_tiling_on_sc=False),
        scratch_types=scratch,
    )


def _split_out(ref, val):
    dh = val.shape[-1] // 2
    ref[0] = val[:, :dh]
    ref[1] = val[:, dh:]


def _cat(ref):
    return jnp.concatenate([ref[0], ref[1]], axis=-1)


def _enc_body(x, ew, eb, eg, ebeta, lw, lb, h_out, xlin_out):
    h = jnp.maximum(_ln(x[...] @ ew[...] + eb[...], eg[...], ebeta[...]), 0.0)
    h_out[...] = h
    _split_out(xlin_out, h @ lw[...] + lb[...])


def _edge_term(ea_ref, dg_ref, eenc_w, eenc_b, edge_w, edge_b):
    ea = ea_ref[...]
    deg = dg_ref[...][:, 0:1]
    ef = ea @ eenc_w[...] + deg * eenc_b[...]
    return ef @ edge_w[...] + (deg + 1.0) * edge_b[...]


def _mid_body(h, acc, xlin, ea, dg, eenc_w, eenc_b, edge_w, edge_b,
              self_w, self_b, skip_w, skip_b, ln_g, ln_b, nlw, nlb,
              hn_out, xlinn_out, *, has_skip):
    hv = h[...]
    aggr = (_cat(acc) + _cat(xlin)
            + _edge_term(ea, dg, eenc_w, eenc_b, edge_w, edge_b))
    hout = jnp.maximum(aggr, 0.0) + hv @ self_w[...] + self_b[...]
    if has_skip:
        hout = hout + hv @ skip_w[...] + skip_b[...]
    hn = jnp.maximum(_ln(hout, ln_g[...], ln_b[...]), 0.0)
    hn_out[...] = hn
    _split_out(xlinn_out, hn @ nlw[...] + nlb[...])


def _lastpool_body(h, acc, xlin, ea, dg, batch, eenc_w, eenc_b, edge_w,
                   edge_b, self_w, self_b, skip_w, skip_b, ln_g, ln_b,
                   c1w, c1b, c1g, c1be, c2w, c2b, c2g, c2be, c3w, c3b,
                   out, sums_sc, cnts_sc, *, num_graphs, nsteps):
    i = pl.program_id(0)
    hv = h[...]
    aggr = (_cat(acc) + _cat(xlin)
            + _edge_term(ea, dg, eenc_w, eenc_b, edge_w, edge_b))
    hout = (jnp.maximum(aggr, 0.0) + hv @ self_w[...] + self_b[...]
            + hv @ skip_w[...] + skip_b[...])
    hn = _ln(hout, ln_g[...], ln_b[...])
    blk = hn.shape[0]
    gids = lax.broadcasted_iota(jnp.int32, (num_graphs, blk), 0)
    mask = (gids == batch[0]).astype(jnp.float32)
    psum = mask @ hn
    pcnt = jnp.broadcast_to(jnp.sum(mask, axis=1, keepdims=True),
                            sums_sc.shape)

    @pl.when(i == 0)
    def _():
        sums_sc[...] = jnp.zeros_like(sums_sc)
        cnts_sc[...] = jnp.zeros_like(cnts_sc)

    sums_sc[...] += psum
    cnts_sc[...] += pcnt

    @pl.when(i == nsteps - 1)
    def _():
        graph = sums_sc[...] / jnp.maximum(cnts_sc[...], 1.0)
        z = jnp.maximum(_ln(graph @ c1w[...] + c1b[...],
                            c1g[...], c1be[...]), 0.0)
        z = jnp.maximum(_ln(z @ c2w[...] + c2b[...], c2g[...], c2be[...]),
                        0.0)
        out[...] = z @ c3w[...] + c3b[...]


def _tc_call(body, out_shapes, *args):
    return pl.pallas_call(
        body,
        out_shape=out_shapes,
    )(*args)


def _tc_call_rowblocked(body, n, hd, blk, emit_next, *args):
    grid = (n // blk,)

    def spec_for(a):
        s = a.shape
        if s[-2] == n and len(s) == 2:
            return pl.BlockSpec((blk, s[-1]), lambda i: (i, 0))
        if len(s) == 3 and s[1] == n:
            return pl.BlockSpec((s[0], blk, s[2]), lambda i: (0, i, 0))
        return pl.BlockSpec(s, lambda i: tuple(0 for _ in s))

    out_specs = [pl.BlockSpec((blk, hd), lambda i: (i, 0))]
    out_shape = [jax.ShapeDtypeStruct((n, hd), jnp.float32)]
    if emit_next:
        out_specs.append(pl.BlockSpec((2, blk, hd // 2), lambda i: (0, i, 0)))
        out_shape.append(jax.ShapeDtypeStruct((2, n, hd // 2), jnp.float32))

    return pl.pallas_call(
        body,
        grid=grid,
        in_specs=[spec_for(a) for a in args],
        out_specs=tuple(out_specs) if emit_next else out_specs[0],
        out_shape=tuple(out_shape) if emit_next else out_shape[0],
    )(*args)


def kernel(x, edge_index, edge_attr, batch, params):
    n, df = x.shape
    e = edge_index.shape[1]
    de = edge_attr.shape[1]
    hd = params['enc_w'].shape[1]
    g = 16
    layers = params['layers']
    nl = len(layers)

    src = edge_index[0]
    dst = edge_index[1]
    src2 = src.reshape(e // _CH, _CH)
    dst2 = dst.reshape(e // _CH, _CH)
    zero128 = jnp.zeros((n // 16, hd // 2), jnp.float32)
    zero16 = jnp.zeros((n // 16, de), jnp.float32)
    zero8 = jnp.zeros((n // 16, 8), jnp.float32)
    ones8 = jnp.ones((_CH, 8), jnp.float32)

    h, xlin = _tc_call(
        _enc_body,
        (jax.ShapeDtypeStruct((n, hd), jnp.float32),
         jax.ShapeDtypeStruct((2, n, hd // 2), jnp.float32)),
        x, params['enc_w'], _row(params['enc_b']),
        _row(params['enc_g']), _row(params['enc_beta']),
        layers[0]['lin_w'], _row(layers[0]['lin_b']))

    spmm = _make_spmm(n, e, hd)
    for i in range(nl - 1):
        lp = layers[i]
        nxt = layers[i + 1]
        if i > 0:
            sp = params['skips'][i - 1]
            skw, skb = sp['w'], _row(sp['b'])
        else:
            skw, skb = lp['self_w'], _row(lp['self_b'])
        if i == 0:
            acc, ea_p, dg_p = _make_spmm(n, e, hd, de)(
                xlin, src2, dst2, zero128, edge_attr, ones8, zero16, zero8)
        else:
            acc = spmm(xlin, src2, dst2, zero128)
        h, xlin = _tc_call_rowblocked(
            functools.partial(_mid_body, has_skip=(i > 0)),
            n, hd, 2000, True,
            h, acc, xlin, ea_p, dg_p,
            params['eenc_w'], _row(params['eenc_b']),
            lp['edge_w'], _row(lp['edge_b']),
            lp['self_w'], _row(lp['self_b']), skw, skb,
            _row(lp['ln_g']), _row(lp['ln_b']),
            nxt['lin_w'], _row(nxt['lin_b']))

    lp = layers[nl - 1]
    sp = params['skips'][nl - 2]
    acc = spmm(xlin, src2, dst2, zero128)
    c3w = jnp.zeros((hd // 2, 128), jnp.float32).at[:, :params['c3_w'].shape[1]].set(params['c3_w'])
    c3b = jnp.zeros((1, 128), jnp.float32).at[0, :params['c3_b'].shape[0]].set(params['c3_b'])
    blk = 2000
    nsteps = n // blk
    args = [h, acc, xlin, ea_p, dg_p, batch.reshape(nsteps, 1, blk),
            params['eenc_w'], _row(params['eenc_b']),
            lp['edge_w'], _row(lp['edge_b']),
            lp['self_w'], _row(lp['self_b']),
            sp['w'], _row(sp['b']),
            _row(lp['ln_g']), _row(lp['ln_b']),
            params['c1_w'], _row(params['c1_b']),
            _row(params['c1_g']), _row(params['c1_beta']),
            params['c2_w'], _row(params['c2_b']),
            _row(params['c2_g']), _row(params['c2_beta']),
            c3w, c3b]

    def spec_for(a):
        s = a.shape
        if len(s) == 2 and s[-2] == n:
            return pl.BlockSpec((blk, s[-1]), lambda i: (i, 0))
        if len(s) == 3 and s[1] == n:
            return pl.BlockSpec((s[0], blk, s[2]), lambda i: (0, i, 0))
        if len(s) == 3 and s[1] == 1:
            return pl.BlockSpec((1, 1, s[2]), lambda i: (i, 0, 0))
        return pl.BlockSpec(s, lambda i: tuple(0 for _ in s))

    out = pl.pallas_call(
        functools.partial(_lastpool_body, num_graphs=g, nsteps=nsteps),
        grid=(nsteps,),
        in_specs=[spec_for(a) for a in args],
        out_specs=pl.BlockSpec((g, 128), lambda i: (0, 0)),
        out_shape=jax.ShapeDtypeStruct((g, 128), jnp.float32),
        scratch_shapes=[pltpu.VMEM((g, 128), jnp.float32),
                        pltpu.VMEM((g, 128), jnp.float32)],
    )(*args)
    return out[:, :params['c3_w'].shape[1]]


# speedup vs baseline: 1.0779x; 1.0779x over previous
import functools

import jax
import jax.numpy as jnp
from jax import lax
from jax.experimental import pallas as pl
from jax.experimental.pallas import tpu as pltpu
from jax.experimental.pallas import tpu_sc as plsc


def _ln(x, g, b):
    m = jnp.mean(x, axis=-1, keepdims=True)
    v = jnp.mean((x - m) * (x - m), axis=-1, keepdims=True)
    return (x - m) / jnp.sqrt(v + 1e-5) * g + b


def _row(v):
    return v.reshape(1, -1)


_CH = 80
_KBUF = 5


@functools.lru_cache(maxsize=None)
def _make_spmm(n, e, d):
    dh = d // 2
    epw = e // 16
    ch = _CH
    nchunk = epw // ch
    k = 5
    ngroup = nchunk // k
    rpt = n // 16
    mesh = plsc.VectorSubcoreMesh(core_axis_name="c", subcore_axis_name="s")

    def body(xlin_hbm, src_hbm, dst_hbm, zero_hbm, out_hbm,
             idx_s, idx_d, rows, acc_sh, gsem, ssem):
        cid = lax.axis_index("c")
        sid = lax.axis_index("s")
        c1 = pltpu.async_copy(src_hbm.at[pl.ds(sid * nchunk, nchunk), :],
                              idx_s, gsem.at[0])
        c2 = pltpu.async_copy(dst_hbm.at[pl.ds(sid * nchunk, nchunk), :],
                              idx_d, gsem.at[1])
        c3 = pltpu.async_copy(zero_hbm, acc_sh.at[pl.ds(sid * rpt, rpt)],
                              gsem.at[2])
        c1.wait()
        c2.wait()
        c3.wait()
        plsc.subcore_barrier()
        half = xlin_hbm.at[cid]

        def group(g, carry):
            base = g * k
            for j in range(k):
                @pl.when(g > 0)
                def _():
                    pltpu.make_async_copy(
                        rows.at[j], acc_sh.at[idx_d.at[base + j]],
                        ssem.at[j]).wait()
                pltpu.async_copy(half.at[idx_s.at[base + j]],
                                 rows.at[j], gsem.at[j])
            for j in range(k):
                pltpu.make_async_copy(half.at[idx_s.at[base + j]],
                                      rows.at[j], gsem.at[j]).wait()
                pltpu.async_copy(rows.at[j], acc_sh.at[idx_d.at[base + j]],
                                 ssem.at[j], add=True)
            return carry

        lax.fori_loop(0, ngroup, group, 0)
        for j in range(k):
            pltpu.make_async_copy(rows.at[j], acc_sh.at[idx_d.at[j]],
                                  ssem.at[j]).wait()
        plsc.subcore_barrier()
        pltpu.sync_copy(acc_sh.at[pl.ds(sid * rpt, rpt)],
                        out_hbm.at[cid, pl.ds(sid * rpt, rpt)])

    return pl.kernel(
        body,
        out_type=jax.ShapeDtypeStruct((2, n, dh), jnp.float32),
        mesh=mesh,
        compiler_params=pltpu.CompilerParams(use_tc_tiling_on_sc=False),
        scratch_types=[
            pltpu.VMEM((nchunk, ch), jnp.int32),
            pltpu.VMEM((nchunk, ch), jnp.int32),
            pltpu.VMEM((k, ch, dh), jnp.float32),
            pltpu.VMEM_SHARED((n, dh), jnp.float32),
            pltpu.SemaphoreType.DMA((k,)),
            pltpu.SemaphoreType.DMA((k,)),
        ],
    )


@functools.lru_cache(maxsize=None)
def _make_preagg(n, e, de):
    nw = 32
    epw = e // nw
    ch = _CH
    nchunk = epw // ch
    rpt = n // 16
    mesh = plsc.VectorSubcoreMesh(core_axis_name="c", subcore_axis_name="s")

    k = _KBUF

    def body(ea_hbm, dst_hbm, zero_hbm, ones_hbm, ea_out, dg_out,
             ebuf, ones, idx_d, acc_ea, acc_dg, lsem, s1sem, s2sem):
        cid = lax.axis_index("c")
        sid = lax.axis_index("s")
        wid = sid * 2 + cid
        c1 = pltpu.async_copy(zero_hbm, acc_ea.at[pl.ds(sid * rpt, rpt)],
                              lsem.at[0])
        c2 = pltpu.async_copy(zero_hbm, acc_dg.at[pl.ds(sid * rpt, rpt)],
                              lsem.at[1])
        c3 = pltpu.async_copy(ones_hbm, ones, lsem.at[2])
        c4 = pltpu.async_copy(dst_hbm.at[pl.ds(wid * nchunk, nchunk), :],
                              idx_d, lsem.at[3])
        c1.wait()
        c2.wait()
        c3.wait()
        c4.wait()
        plsc.subcore_barrier()
        base0 = wid * nchunk

        def group(g, carry):
            base = g * k
            for j in range(k):
                @pl.when(g > 0)
                def _():
                    pltpu.make_async_copy(ebuf.at[j],
                                          acc_ea.at[idx_d.at[base + j]],
                                          s1sem.at[j]).wait()
                    pltpu.make_async_copy(ones,
                                          acc_dg.at[idx_d.at[base + j]],
                                          s2sem.at[j]).wait()
                pltpu.async_copy(
                    ea_hbm.at[pl.ds((base0 + base + j) * ch, ch), :],
                    ebuf.at[j], lsem.at[j])
            for j in range(k):
                pltpu.make_async_copy(
                    ea_hbm.at[pl.ds((base0 + base + j) * ch, ch), :],
                    ebuf.at[j], lsem.at[j]).wait()
                pltpu.async_copy(ebuf.at[j], acc_ea.at[idx_d.at[base + j]],
                                 s1sem.at[j], add=True)
                pltpu.async_copy(ones, acc_dg.at[idx_d.at[base + j]],
                                 s2sem.at[j], add=True)
            return carry

        lax.fori_loop(0, nchunk // k, group, 0)
        for j in range(k):
            pltpu.make_async_copy(ebuf.at[j], acc_ea.at[idx_d.at[j]],
                                  s1sem.at[j]).wait()
            pltpu.make_async_copy(ones, acc_dg.at[idx_d.at[j]],
                                  s2sem.at[j]).wait()
        plsc.subcore_barrier()
        pltpu.sync_copy(acc_ea.at[pl.ds(sid * rpt, rpt)],
                        ea_out.at[cid, pl.ds(sid * rpt, rpt)])
        pltpu.sync_copy(acc_dg.at[pl.ds(sid * rpt, rpt)],
                        dg_out.at[cid, pl.ds(sid * rpt, rpt)])

    return pl.kernel(
        body,
        out_type=(jax.ShapeDtypeStruct((2, n, de), jnp.float32),
                  jax.ShapeDtypeStruct((2, n, de), jnp.float32)),
        mesh=mesh,
        compiler_params=pltpu.CompilerParams(use_tc_tiling_on_sc=False),
        scratch_types=[
            pltpu.VMEM((k, ch, de), jnp.float32),
            pltpu.VMEM((ch, de), jnp.float32),
            pltpu.VMEM((nchunk, ch), jnp.int32),
            pltpu.VMEM_SHARED((n, de), jnp.float32),
            pltpu.VMEM_SHARED((n, de), jnp.float32),
            pltpu.SemaphoreType.DMA((k,)),
            pltpu.SemaphoreType.DMA((k,)),
            pltpu.SemaphoreType.DMA((k,)),
        ],
    )


def _split_out(ref, val):
    dh = val.shape[-1] // 2
    ref[0] = val[:, :dh]
    ref[1] = val[:, dh:]


def _cat(ref):
    return jnp.concatenate([ref[0], ref[1]], axis=-1)


def _enc_body(x, ew, eb, eg, ebeta, lw, lb, h_out, xlin_out):
    h = jnp.maximum(_ln(x[...] @ ew[...] + eb[...], eg[...], ebeta[...]), 0.0)
    h_out[...] = h
    _split_out(xlin_out, h @ lw[...] + lb[...])


def _edge_term(ea_ref, dg_ref, eenc_w, eenc_b, edge_w, edge_b):
    ea = ea_ref[0] + ea_ref[1]
    deg = dg_ref[0][:, 0:1] + dg_ref[1][:, 0:1]
    ef = ea @ eenc_w[...] + deg * eenc_b[...]
    return ef @ edge_w[...] + (deg + 1.0) * edge_b[...]


def _mid_body(h, acc, xlin, ea, dg, eenc_w, eenc_b, edge_w, edge_b,
              self_w, self_b, skip_w, skip_b, ln_g, ln_b, nlw, nlb,
              hn_out, xlinn_out, *, has_skip):
    hv = h[...]
    aggr = (_cat(acc) + _cat(xlin)
            + _edge_term(ea, dg, eenc_w, eenc_b, edge_w, edge_b))
    hout = jnp.maximum(aggr, 0.0) + hv @ self_w[...] + self_b[...]
    if has_skip:
        hout = hout + hv @ skip_w[...] + skip_b[...]
    hn = jnp.maximum(_ln(hout, ln_g[...], ln_b[...]), 0.0)
    hn_out[...] = hn
    _split_out(xlinn_out, hn @ nlw[...] + nlb[...])


def _lastpool_body(h, acc, xlin, ea, dg, batch, eenc_w, eenc_b, edge_w,
                   edge_b, self_w, self_b, skip_w, skip_b, ln_g, ln_b,
                   c1w, c1b, c1g, c1be, c2w, c2b, c2g, c2be, c3w, c3b,
                   out, sums_sc, cnts_sc, *, num_graphs, nsteps):
    i = pl.program_id(0)
    hv = h[...]
    aggr = (_cat(acc) + _cat(xlin)
            + _edge_term(ea, dg, eenc_w, eenc_b, edge_w, edge_b))
    hout = (jnp.maximum(aggr, 0.0) + hv @ self_w[...] + self_b[...]
            + hv @ skip_w[...] + skip_b[...])
    hn = _ln(hout, ln_g[...], ln_b[...])
    blk = hn.shape[0]
    gids = lax.broadcasted_iota(jnp.int32, (num_graphs, blk), 0)
    mask = (gids == batch[0]).astype(jnp.float32)
    psum = mask @ hn
    pcnt = jnp.broadcast_to(jnp.sum(mask, axis=1, keepdims=True),
                            sums_sc.shape)

    @pl.when(i == 0)
    def _():
        sums_sc[...] = jnp.zeros_like(sums_sc)
        cnts_sc[...] = jnp.zeros_like(cnts_sc)

    sums_sc[...] += psum
    cnts_sc[...] += pcnt

    @pl.when(i == nsteps - 1)
    def _():
        graph = sums_sc[...] / jnp.maximum(cnts_sc[...], 1.0)
        z = jnp.maximum(_ln(graph @ c1w[...] + c1b[...],
                            c1g[...], c1be[...]), 0.0)
        z = jnp.maximum(_ln(z @ c2w[...] + c2b[...], c2g[...], c2be[...]),
                        0.0)
        out[...] = z @ c3w[...] + c3b[...]


def _tc_call(body, out_shapes, *args):
    return pl.pallas_call(
        body,
        out_shape=out_shapes,
    )(*args)


def _tc_call_rowblocked(body, n, hd, blk, emit_next, *args):
    grid = (n // blk,)

    def spec_for(a):
        s = a.shape
        if s[-2] == n and len(s) == 2:
            return pl.BlockSpec((blk, s[-1]), lambda i: (i, 0))
        if len(s) == 3 and s[1] == n:
            return pl.BlockSpec((s[0], blk, s[2]), lambda i: (0, i, 0))
        return pl.BlockSpec(s, lambda i: tuple(0 for _ in s))

    out_specs = [pl.BlockSpec((blk, hd), lambda i: (i, 0))]
    out_shape = [jax.ShapeDtypeStruct((n, hd), jnp.float32)]
    if emit_next:
        out_specs.append(pl.BlockSpec((2, blk, hd // 2), lambda i: (0, i, 0)))
        out_shape.append(jax.ShapeDtypeStruct((2, n, hd // 2), jnp.float32))

    return pl.pallas_call(
        body,
        grid=grid,
        in_specs=[spec_for(a) for a in args],
        out_specs=tuple(out_specs) if emit_next else out_specs[0],
        out_shape=tuple(out_shape) if emit_next else out_shape[0],
    )(*args)


def kernel(x, edge_index, edge_attr, batch, params):
    n, df = x.shape
    e = edge_index.shape[1]
    de = edge_attr.shape[1]
    hd = params['enc_w'].shape[1]
    g = 16
    layers = params['layers']
    nl = len(layers)

    src = edge_index[0]
    dst = edge_index[1]
    src2 = src.reshape(e // _CH, _CH)
    dst2 = dst.reshape(e // _CH, _CH)
    zero128 = jnp.zeros((n // 16, hd // 2), jnp.float32)
    zero16 = jnp.zeros((n // 16, de), jnp.float32)
    ones16 = jnp.ones((_CH, de), jnp.float32)

    ea_p, dg_p = _make_preagg(n, e, de)(edge_attr, dst2, zero16, ones16)

    h, xlin = _tc_call(
        _enc_body,
        (jax.ShapeDtypeStruct((n, hd), jnp.float32),
         jax.ShapeDtypeStruct((2, n, hd // 2), jnp.float32)),
        x, params['enc_w'], _row(params['enc_b']),
        _row(params['enc_g']), _row(params['enc_beta']),
        layers[0]['lin_w'], _row(layers[0]['lin_b']))

    spmm = _make_spmm(n, e, hd)
    for i in range(nl - 1):
        lp = layers[i]
        nxt = layers[i + 1]
        if i > 0:
            sp = params['skips'][i - 1]
            skw, skb = sp['w'], _row(sp['b'])
        else:
            skw, skb = lp['self_w'], _row(lp['self_b'])
        acc = spmm(xlin, src2, dst2, zero128)
        h, xlin = _tc_call_rowblocked(
            functools.partial(_mid_body, has_skip=(i > 0)),
            n, hd, 2000, True,
            h, acc, xlin, ea_p, dg_p,
            params['eenc_w'], _row(params['eenc_b']),
            lp['edge_w'], _row(lp['edge_b']),
            lp['self_w'], _row(lp['self_b']), skw, skb,
            _row(lp['ln_g']), _row(lp['ln_b']),
            nxt['lin_w'], _row(nxt['lin_b']))

    lp = layers[nl - 1]
    sp = params['skips'][nl - 2]
    acc = spmm(xlin, src2, dst2, zero128)
    c3w = jnp.zeros((hd // 2, 128), jnp.float32).at[:, :params['c3_w'].shape[1]].set(params['c3_w'])
    c3b = jnp.zeros((1, 128), jnp.float32).at[0, :params['c3_b'].shape[0]].set(params['c3_b'])
    blk = 2000
    nsteps = n // blk
    args = [h, acc, xlin, ea_p, dg_p, batch.reshape(nsteps, 1, blk),
            params['eenc_w'], _row(params['eenc_b']),
            lp['edge_w'], _row(lp['edge_b']),
            lp['self_w'], _row(lp['self_b']),
            sp['w'], _row(sp['b']),
            _row(lp['ln_g']), _row(lp['ln_b']),
            params['c1_w'], _row(params['c1_b']),
            _row(params['c1_g']), _row(params['c1_beta']),
            params['c2_w'], _row(params['c2_b']),
            _row(params['c2_g']), _row(params['c2_beta']),
            c3w, c3b]

    def spec_for(a):
        s = a.shape
        if len(s) == 2 and s[-2] == n:
            return pl.BlockSpec((blk, s[-1]), lambda i: (i, 0))
        if len(s) == 3 and s[1] == n:
            return pl.BlockSpec((s[0], blk, s[2]), lambda i: (0, i, 0))
        if len(s) == 3 and s[1] == 1:
            return pl.BlockSpec((1, 1, s[2]), lambda i: (i, 0, 0))
        return pl.BlockSpec(s, lambda i: tuple(0 for _ in s))

    out = pl.pallas_call(
        functools.partial(_lastpool_body, num_graphs=g, nsteps=nsteps),
        grid=(nsteps,),
        in_specs=[spec_for(a) for a in args],
        out_specs=pl.BlockSpec((g, 128), lambda i: (0, 0)),
        out_shape=jax.ShapeDtypeStruct((g, 128), jnp.float32),
        scratch_shapes=[pltpu.VMEM((g, 128), jnp.float32),
                        pltpu.VMEM((g, 128), jnp.float32)],
    )(*args)
    return out[:, :params['c3_w'].shape[1]]
